# baseline (speedup 1.0000x reference)
import functools
import numpy as np
import jax
from jax import lax
import jax.numpy as jnp
from jax.experimental import pallas as pl
from jax.experimental.pallas import tpu as pltpu
from jax.experimental.pallas import tpu_sc as plsc

_SLOW_FRAMES = 8


def _slow_indices(t):
    return tuple(int(v) for v in np.linspace(0, t - 1, _SLOW_FRAMES).astype(np.int32))


def kernel(frames):
    b, t, c, h, w = frames.shape
    n_slow = _SLOW_FRAMES
    assert _slow_indices(t) == tuple((j * (t - 1)) // (n_slow - 1) for j in range(n_slow))
    mesh = plsc.VectorSubcoreMesh(core_axis_name="c", subcore_axis_name="s")
    n_workers = 32
    chunks = b * n_slow
    per_w = chunks // n_workers

    @functools.partial(
        pl.kernel,
        mesh=mesh,
        out_type=jax.ShapeDtypeStruct((b, n_slow, c, h, w), frames.dtype),
        scratch_types=[
            pltpu.SemaphoreType.DMA,
            pltpu.SemaphoreType.DMA,
        ],
    )
    def sc_gather(frames_hbm, slow_hbm, sem0, sem1):
        cid = lax.axis_index("c")
        sid = lax.axis_index("s")
        wid = sid * 2 + cid
        sems = (sem0, sem1)
        cps = []
        for p in range(per_w):
            r = wid * per_w + p
            bi = r // n_slow
            j = r % n_slow
            ti = (j * (t - 1)) // (n_slow - 1)
            cp = pltpu.make_async_copy(
                frames_hbm.at[bi, ti], slow_hbm.at[bi, j], sems[p % 2]
            )
            cp.start()
            cps.append(cp)
        for cp in cps:
            cp.wait()

    slow = sc_gather(frames)
    return (slow, frames)


# speedup vs baseline: 9.3064x; 9.3064x over previous
import functools
import numpy as np
import jax
from jax import lax
import jax.numpy as jnp
from jax.experimental import pallas as pl
from jax.experimental.pallas import tpu as pltpu
from jax.experimental.pallas import tpu_sc as plsc

_SLOW_FRAMES = 8
_PH = 64
_K = 6
_LEAD = 3


def _slow_indices(t):
    return tuple(int(v) for v in np.linspace(0, t - 1, _SLOW_FRAMES).astype(np.int32))


def kernel(frames):
    b, t, c, h, w = frames.shape
    n_slow = _SLOW_FRAMES
    assert _slow_indices(t) == tuple((j * (t - 1)) // (n_slow - 1) for j in range(n_slow))
    mesh = plsc.VectorSubcoreMesh(core_axis_name="c", subcore_axis_name="s")
    n_workers = 32
    chunks = b * n_slow
    per_w = chunks // n_workers
    pp_h = h // _PH
    ppc = c * pp_h
    n_pieces = per_w * ppc

    @functools.partial(
        pl.kernel,
        mesh=mesh,
        out_type=jax.ShapeDtypeStruct((b, n_slow, c, h, w), frames.dtype),
        scratch_types=(
            [pltpu.VMEM((_PH, w), frames.dtype)] * _K
            + [pltpu.SemaphoreType.DMA] * (2 * _K)
        ),
    )
    def sc_gather(frames_hbm, slow_hbm, *scratch):
        bufs = scratch[:_K]
        in_sems = scratch[_K : 2 * _K]
        out_sems = scratch[2 * _K : 3 * _K]
        cid = lax.axis_index("c")
        sid = lax.axis_index("s")
        wid = sid * 2 + cid

        def coords(p):
            r = wid * per_w + p // ppc
            q = p % ppc
            ci, hp = q // pp_h, q % pp_h
            bi = r // n_slow
            j = r % n_slow
            ti = (j * (t - 1)) // (n_slow - 1)
            return bi, j, ti, ci, hp * _PH

        ins, outs = {}, {}
        for step in range(n_pieces + _LEAD):
            if step < n_pieces:
                k = step % _K
                bi, j, ti, ci, row0 = coords(step)
                if step >= _K:
                    outs[step - _K].wait()
                cp = pltpu.make_async_copy(
                    frames_hbm.at[bi, ti, ci, pl.ds(row0, _PH)], bufs[k], in_sems[k]
                )
                cp.start()
                ins[step] = cp
            r = step - _LEAD
            if r >= 0:
                k = r % _K
                bi, j, ti, ci, row0 = coords(r)
                ins[r].wait()
                cp = pltpu.make_async_copy(
                    bufs[k], slow_hbm.at[bi, j, ci, pl.ds(row0, _PH)], out_sems[k]
                )
                cp.start()
                outs[r] = cp
        for r in range(max(0, n_pieces - _K), n_pieces):
            outs[r].wait()

    slow = sc_gather(frames)
    return (slow, frames)


# speedup vs baseline: 9.3135x; 1.0008x over previous
import functools
import numpy as np
import jax
from jax import lax
import jax.numpy as jnp
from jax.experimental import pallas as pl
from jax.experimental.pallas import tpu as pltpu
from jax.experimental.pallas import tpu_sc as plsc

_SLOW_FRAMES = 8
_PH = 64
_K = 7
_LEAD = 3


def _slow_indices(t):
    return tuple(int(v) for v in np.linspace(0, t - 1, _SLOW_FRAMES).astype(np.int32))


def kernel(frames):
    b, t, c, h, w = frames.shape
    n_slow = _SLOW_FRAMES
    assert _slow_indices(t) == tuple((j * (t - 1)) // (n_slow - 1) for j in range(n_slow))
    mesh = plsc.VectorSubcoreMesh(core_axis_name="c", subcore_axis_name="s")
    n_workers = 32
    chunks = b * n_slow
    per_w = chunks // n_workers
    pp_h = h // _PH
    ppc = c * pp_h
    n_pieces = per_w * ppc

    @functools.partial(
        pl.kernel,
        mesh=mesh,
        out_type=jax.ShapeDtypeStruct((b, n_slow, c, h, w), frames.dtype),
        scratch_types=(
            [pltpu.VMEM((_PH, w), frames.dtype)] * _K
            + [pltpu.SemaphoreType.DMA] * (2 * _K)
        ),
    )
    def sc_gather(frames_hbm, slow_hbm, *scratch):
        bufs = scratch[:_K]
        in_sems = scratch[_K : 2 * _K]
        out_sems = scratch[2 * _K : 3 * _K]
        cid = lax.axis_index("c")
        sid = lax.axis_index("s")
        wid = sid * 2 + cid

        def coords(p):
            r = wid * per_w + p // ppc
            q = p % ppc
            ci, hp = q // pp_h, q % pp_h
            bi = r // n_slow
            j = r % n_slow
            ti = (j * (t - 1)) // (n_slow - 1)
            return bi, j, ti, ci, hp * _PH

        ins, outs = {}, {}
        for step in range(n_pieces + _LEAD):
            if step < n_pieces:
                k = step % _K
                bi, j, ti, ci, row0 = coords(step)
                if step >= _K:
                    outs[step - _K].wait()
                cp = pltpu.make_async_copy(
                    frames_hbm.at[bi, ti, ci, pl.ds(row0, _PH)], bufs[k], in_sems[k]
                )
                cp.start()
                ins[step] = cp
            r = step - _LEAD
            if r >= 0:
                k = r % _K
                bi, j, ti, ci, row0 = coords(r)
                ins[r].wait()
                cp = pltpu.make_async_copy(
                    bufs[k], slow_hbm.at[bi, j, ci, pl.ds(row0, _PH)], out_sems[k]
                )
                cp.start()
                outs[r] = cp
        for r in range(max(0, n_pieces - _K), n_pieces):
            outs[r].wait()

    slow = sc_gather(frames)
    return (slow, frames)


# speedup vs baseline: 9.3282x; 1.0016x over previous
import functools
import numpy as np
import jax
from jax import lax
import jax.numpy as jnp
from jax.experimental import pallas as pl
from jax.experimental.pallas import tpu as pltpu
from jax.experimental.pallas import tpu_sc as plsc

_SLOW_FRAMES = 8
_KA = 4
_KB = 2
_PA = 64
_PB = 128


def _slow_indices(t):
    return tuple(int(v) for v in np.linspace(0, t - 1, _SLOW_FRAMES).astype(np.int32))


def kernel(frames):
    b, t, c, h, w = frames.shape
    n_slow = _SLOW_FRAMES
    assert _slow_indices(t) == tuple((j * (t - 1)) // (n_slow - 1) for j in range(n_slow))
    mesh = plsc.VectorSubcoreMesh(core_axis_name="c", subcore_axis_name="s")
    n_workers = 32
    chunks = b * n_slow
    per_w = chunks // n_workers

    @functools.partial(
        pl.kernel,
        mesh=mesh,
        out_type=jax.ShapeDtypeStruct((b, n_slow, c, h, w), frames.dtype),
        scratch_types=(
            [pltpu.VMEM((_PA, w), frames.dtype)] * _KA
            + [pltpu.VMEM_SHARED((16, _KB, _PB, w), frames.dtype)]
            + [pltpu.SemaphoreType.DMA] * (2 * _KA + 2 * _KB)
        ),
    )
    def sc_gather(frames_hbm, slow_hbm, *scratch):
        bufs_a = scratch[:_KA]
        shared = scratch[_KA]
        sems = scratch[_KA + 1 :]
        in_a = sems[:_KA]
        out_a = sems[_KA : 2 * _KA]
        in_b = sems[2 * _KA : 2 * _KA + _KB]
        out_b = sems[2 * _KA + _KB :]
        cid = lax.axis_index("c")
        sid = lax.axis_index("s")
        wid = sid * 2 + cid

        def chunk_coords(m):
            r = wid * per_w + m
            bi = r // n_slow
            j = r % n_slow
            ti = (j * (t - 1)) // (n_slow - 1)
            return bi, j, ti

        pieces_a = []
        for m in range(per_w):
            for ci in range(c):
                for hp in range(2):
                    pieces_a.append((m, ci, hp * _PA))
        pieces_b = []
        for m in range(per_w):
            for ci in range(c):
                pieces_b.append((m, ci, _PB))

        class Ring:
            def __init__(self, K, lead, pieces, ph, bufs, in_sems, out_sems):
                self.K, self.lead, self.pieces, self.ph = K, lead, pieces, ph
                self.bufs, self.in_sems, self.out_sems = bufs, in_sems, out_sems
                self.step = 0
                self.ins, self.outs = {}, {}

            def done(self):
                return self.step >= len(self.pieces) + self.lead

            def advance(self):
                s = self.step
                n = len(self.pieces)
                if s < n:
                    k = s % self.K
                    m, ci, row0 = self.pieces[s]
                    bi, j, ti = chunk_coords(m)
                    if s >= self.K:
                        self.outs[s - self.K].wait()
                    cp = pltpu.make_async_copy(
                        frames_hbm.at[bi, ti, ci, pl.ds(row0, self.ph)],
                        self.bufs[k],
                        self.in_sems[k],
                    )
                    cp.start()
                    self.ins[s] = cp
                r = s - self.lead
                if 0 <= r < n:
                    k = r % self.K
                    m, ci, row0 = self.pieces[r]
                    bi, j, ti = chunk_coords(m)
                    self.ins[r].wait()
                    cp = pltpu.make_async_copy(
                        self.bufs[k],
                        slow_hbm.at[bi, j, ci, pl.ds(row0, self.ph)],
                        self.out_sems[k],
                    )
                    cp.start()
                    self.outs[r] = cp
                self.step += 1

            def drain(self):
                n = len(self.pieces)
                for r in range(max(0, n - self.K), n):
                    self.outs[r].wait()

        bufs_b = [shared.at[sid, kb] for kb in range(_KB)]
        ring_a = Ring(_KA, 2, pieces_a, _PA, bufs_a, in_a, out_a)
        ring_b = Ring(_KB, 1, pieces_b, _PB, bufs_b, in_b, out_b)

        while not (ring_a.done() and ring_b.done()):
            if not ring_a.done():
                ring_a.advance()
            if not ring_a.done():
                ring_a.advance()
            if not ring_b.done():
                ring_b.advance()
        ring_a.drain()
        ring_b.drain()

    slow = sc_gather(frames)
    return (slow, frames)
